# Initial kernel scaffold; baseline (speedup 1.0000x reference)
import jax
import jax.numpy as jnp
from jax.experimental import pallas as pl


def kernel(dense_features1, dense_features2, img_ref_hr, vgg_params):
    raise NotImplementedError("write your pallas kernel here")


# speedup vs baseline: 6.8889x; 6.8889x over previous
import functools
import jax
import jax.numpy as jnp
from jax.experimental import pallas as pl
from jax.experimental.pallas import tpu as pltpu

_PS = 3
_TOPK = 3
_H = 48
_HW = _H * _H
_PAD = 2432
_C = 256
_K9 = _C * _PS * _PS
_MT = 128
_NT = 384
_NMI = _HW // _MT
_NNI = _HW // _NT
_VALID = _H - _PS + 1
_NEG = -1.0e30
_INIT = -3.0e38


def _corr_topk_body(fi_ref, fr_ref, val_ref, idx_ref,
                    b9, a9t, inv_i, inv_r, nin_inv, rv, ri):
    mi = pl.program_id(1)
    ni = pl.program_id(2)

    @pl.when(jnp.logical_and(mi == 0, ni == 0))
    def _batch_init():
        for dx in range(_PS):
            fi = fi_ref[0, dx]
            n = jnp.sqrt(jnp.sum(fi * fi, axis=1, keepdims=True))
            inv_i[:, dx:dx + 1] = 1.0 / jnp.maximum(n, 1e-12)
            fr = fr_ref[0, dx]
            n2 = jnp.sqrt(jnp.sum(fr * fr, axis=1, keepdims=True))
            inv_r[:, dx:dx + 1] = 1.0 / jnp.maximum(n2, 1e-12)
        for dy in range(_PS):
            for dx in range(_PS):
                tap = dy * _PS + dx
                b9[:, pl.ds(tap * _C, _C)] = (
                    fr_ref[0, dx, pl.ds(dy * _H, _HW), :]
                    * inv_r[pl.ds(dy * _H, _HW), dx:dx + 1])
        bn = b9[...]
        rn = jnp.sqrt(jnp.sum(bn * bn, axis=1, keepdims=True))
        b9[...] = bn * (1.0 / (rn + 1e-5))

    @pl.when(ni == 0)
    def _mtile_init():
        m0 = pl.multiple_of(mi * _MT, _MT)
        for dy in range(_PS):
            for dx in range(_PS):
                tap = dy * _PS + dx
                a9t[:, pl.ds(tap * _C, _C)] = (
                    fi_ref[0, dx, pl.ds(m0 + dy * _H, _MT), :]
                    * inv_i[pl.ds(m0 + dy * _H, _MT), dx:dx + 1])
        a = a9t[...]
        nin2 = jnp.sum(a * a, axis=1, keepdims=True)
        nin_inv[...] = 1.0 / (jnp.sqrt(nin2) + 1e-5)
        rv[...] = jnp.full((_MT, 8), _INIT, jnp.float32)
        ri[...] = jnp.zeros((_MT, 8), jnp.int32)

    c = jax.lax.dot_general(a9t[...], b9[pl.ds(ni * _NT, _NT), :],
                            (((1,), (1,)), ((), ())),
                            preferred_element_type=jnp.float32)
    p48 = ni * _NT + jax.lax.broadcasted_iota(jnp.int32, (_MT, _NT), 1)
    bad = jnp.logical_or(p48 % _H >= _VALID, p48 >= _VALID * _H)
    c = jnp.where(bad, _NEG, c)

    for _ in range(_TOPK):
        m = jnp.max(c, axis=1, keepdims=True)
        sel = jnp.min(jnp.where(c >= m, p48, jnp.int32(2 ** 30)),
                      axis=1, keepdims=True)
        c = jnp.where(p48 == sel, _NEG, c)
        r0, r1, r2 = rv[:, 0:1], rv[:, 1:2], rv[:, 2:3]
        i0, i1, i2 = ri[:, 0:1], ri[:, 1:2], ri[:, 2:3]
        b0 = m > r0
        b1 = m > r1
        b2 = m > r2
        rv[:, 2:3] = jnp.where(b1, r1, jnp.where(b2, m, r2))
        ri[:, 2:3] = jnp.where(b1, i1, jnp.where(b2, sel, i2))
        rv[:, 1:2] = jnp.where(b0, r0, jnp.where(b1, m, r1))
        ri[:, 1:2] = jnp.where(b0, i0, jnp.where(b1, sel, i1))
        rv[:, 0:1] = jnp.where(b0, m, r0)
        ri[:, 0:1] = jnp.where(b0, sel, i0)

    @pl.when(ni == _NNI - 1)
    def _emit():
        val_ref[0] = rv[...] * nin_inv[...]
        idx_ref[0] = ri[...]


def _corr_topk(fi_dx, fr_dx):
    B = fi_dx.shape[0]
    grid = (B, _NMI, _NNI)
    return pl.pallas_call(
        _corr_topk_body,
        grid=grid,
        in_specs=[
            pl.BlockSpec((1, _PS, _PAD, _C), lambda b, mi, ni: (b, 0, 0, 0)),
            pl.BlockSpec((1, _PS, _PAD, _C), lambda b, mi, ni: (b, 0, 0, 0)),
        ],
        out_specs=[
            pl.BlockSpec((1, _MT, 8), lambda b, mi, ni: (b, mi, 0)),
            pl.BlockSpec((1, _MT, 8), lambda b, mi, ni: (b, mi, 0)),
        ],
        out_shape=[
            jax.ShapeDtypeStruct((B, _HW, 8), jnp.float32),
            jax.ShapeDtypeStruct((B, _HW, 8), jnp.int32),
        ],
        compiler_params=pltpu.CompilerParams(
            vmem_limit_bytes=100 * 1024 * 1024,
        ),
        scratch_shapes=[
            pltpu.VMEM((_HW, _K9), jnp.float32),
            pltpu.VMEM((_MT, _K9), jnp.float32),
            pltpu.VMEM((_PAD, 8), jnp.float32),
            pltpu.VMEM((_PAD, 8), jnp.float32),
            pltpu.VMEM((_MT, 1), jnp.float32),
            pltpu.VMEM((_MT, 8), jnp.float32),
            pltpu.VMEM((_MT, 8), jnp.int32),
        ],
    )(fi_dx, fr_dx)


def _shift(x, i, j):
    if i == 0 and j == 0:
        return x
    nd = x.ndim
    pad = [(0, 0)] * nd
    pad[-3] = (i, 0)
    pad[-2] = (j, 0)
    h, w = x.shape[-3], x.shape[-2]
    return jnp.pad(x, pad)[..., :h, :w, :]


def _up2(x, axis_h, axis_w):
    return jnp.repeat(jnp.repeat(x, 2, axis=axis_h), 2, axis=axis_w)


def _vgg(img, p):
    mean = jnp.array([0.485, 0.456, 0.406], jnp.float32).reshape(1, 3, 1, 1)
    std = jnp.array([0.229, 0.224, 0.225], jnp.float32).reshape(1, 3, 1, 1)
    x = (img - mean) / std

    def conv(x, w, b):
        y = jax.lax.conv_general_dilated(x, w, (1, 1), 'SAME',
                                         dimension_numbers=('NCHW', 'OIHW', 'NCHW'))
        return y + b[None, :, None, None]

    def pool(x):
        return jax.lax.reduce_window(x, -jnp.inf, jax.lax.max,
                                     (1, 1, 2, 2), (1, 1, 2, 2), 'VALID')

    x = jax.nn.relu(conv(x, p['w11'], p['b11'])); r1 = x
    x = jax.nn.relu(conv(x, p['w12'], p['b12']))
    x = pool(x)
    x = jax.nn.relu(conv(x, p['w21'], p['b21'])); r2 = x
    x = jax.nn.relu(conv(x, p['w22'], p['b22']))
    x = pool(x)
    x = jax.nn.relu(conv(x, p['w31'], p['b31'])); r3 = x
    return r1, r2, r3


@jax.jit
def _run(dense_features1, dense_features2, img_ref_hr, vgg_params):
    B = dense_features1.shape[0]

    def prep(x):
        xt = jnp.transpose(x.reshape(B, _C, _HW), (0, 2, 1))
        xp = jnp.pad(xt, ((0, 0), (0, _PAD + 2 - _HW), (0, 0)))
        return jnp.stack([xp[:, dx:dx + _PAD, :] for dx in range(_PS)], axis=1)

    vals, idxs = _corr_topk(prep(dense_features1), prep(dense_features2))

    v = vals[:, :, :_TOPK].reshape(B, _H, _H, _TOPK)[:, :_VALID, :_VALID, :]
    v = jnp.transpose(v, (0, 3, 1, 2))
    ix = idxs[:, :, :_TOPK].reshape(B, _H, _H, _TOPK)[:, :_VALID, :_VALID, :]
    ix = jnp.transpose(ix, (0, 3, 1, 2))

    u = (ix % _H).astype(jnp.float32)
    vv = (ix // _H).astype(jnp.float32)
    gy = jnp.arange(_VALID, dtype=jnp.float32)[None, None, :, None]
    gx = jnp.arange(_VALID, dtype=jnp.float32)[None, None, None, :]
    flow = jnp.stack((u - gx, vv - gy), axis=-1)
    kf3 = jnp.pad(flow, ((0, 0), (0, 0), (0, 2), (0, 2), (0, 0)))
    ks3 = jnp.pad(v, ((0, 0), (0, 0), (1, 1), (1, 1)))

    ko3 = jnp.stack([_shift(kf3, i, j) for i in range(3) for j in range(3)], axis=2)
    kf2 = _up2(kf3, 2, 3) * 2.0
    ks2 = _up2(ks3, 2, 3)
    ko2 = jnp.stack([_shift(kf2, 2 * i, 2 * j) for i in range(3) for j in range(3)], axis=2)
    kf1 = _up2(_up2(kf3, 2, 3), 2, 3) * 4.0
    ks1 = _up2(_up2(ks3, 2, 3), 2, 3)
    ko1 = jnp.stack([_shift(kf1, 4 * i, 4 * j) for i in range(3) for j in range(3)], axis=2)

    r1, r2, r3 = _vgg(img_ref_hr, vgg_params)
    return (kf1, kf2, kf3, ko1, ko2, ko3, ks1, ks2, ks3, r1, r2, r3)


def kernel(dense_features1, dense_features2, img_ref_hr, vgg_params):
    return _run(dense_features1, dense_features2, img_ref_hr, vgg_params)


# speedup vs baseline: 8.5275x; 1.2379x over previous
import functools
import jax
import jax.numpy as jnp
from jax.experimental import pallas as pl
from jax.experimental.pallas import tpu as pltpu

_PS = 3
_TOPK = 3
_H = 48
_HW = _H * _H
_PAD = 2432
_C = 256
_K9 = _C * _PS * _PS
_MT = 128
_NT = 768
_NMI = _HW // _MT
_NNI = _HW // _NT
_VALID = _H - _PS + 1
_NEG = -1.0e30
_INIT = -3.0e38


def _corr_topk_body(fi_ref, fr_ref, val_ref, idx_ref,
                    b9, a9t, inv_i, inv_r, qr_i, nrm_inv, nin_inv, rv, ri):
    mi = pl.program_id(1)
    ni = pl.program_id(2)

    @pl.when(jnp.logical_and(mi == 0, ni == 0))
    def _batch_init():
        for dx in range(_PS):
            fi = fi_ref[0, dx]
            n2 = jnp.sum(fi * fi, axis=1, keepdims=True)
            n = jnp.sqrt(n2)
            iv = 1.0 / jnp.maximum(n, 1e-12)
            inv_i[:, dx:dx + 1] = iv
            qr_i[:, dx:dx + 1] = n2 * iv * iv
            fr = fr_ref[0, dx]
            m2 = jnp.sum(fr * fr, axis=1, keepdims=True)
            m = jnp.sqrt(m2)
            ivr = 1.0 / jnp.maximum(m, 1e-12)
            inv_r[:, dx:dx + 1] = ivr
            qr_i[:, 4 + dx:5 + dx] = m2 * ivr * ivr
        s = jnp.zeros((_HW, 1), jnp.float32)
        for dy in range(_PS):
            for dx in range(_PS):
                s = s + qr_i[pl.ds(dy * _H, _HW), 4 + dx:5 + dx]
        nrm_inv[...] = 1.0 / (jnp.sqrt(s) + 1e-5)
        niv = nrm_inv[...]
        for dy in range(_PS):
            for dx in range(_PS):
                tap = dy * _PS + dx
                b9[:, pl.ds(tap * _C, _C)] = (
                    fr_ref[0, dx, pl.ds(dy * _H, _HW), :]
                    * (inv_r[pl.ds(dy * _H, _HW), dx:dx + 1] * niv))

    @pl.when(ni == 0)
    def _mtile_init():
        m0 = pl.multiple_of(mi * _MT, _MT)
        for dy in range(_PS):
            for dx in range(_PS):
                tap = dy * _PS + dx
                a9t[:, pl.ds(tap * _C, _C)] = (
                    fi_ref[0, dx, pl.ds(m0 + dy * _H, _MT), :]
                    * inv_i[pl.ds(m0 + dy * _H, _MT), dx:dx + 1])
        nin2 = jnp.zeros((_MT, 1), jnp.float32)
        for dy in range(_PS):
            for dx in range(_PS):
                nin2 = nin2 + qr_i[pl.ds(m0 + dy * _H, _MT), dx:dx + 1]
        nin_inv[...] = 1.0 / (jnp.sqrt(nin2) + 1e-5)
        rv[...] = jnp.full((_MT, 8), _INIT, jnp.float32)
        ri[...] = jnp.zeros((_MT, 8), jnp.int32)

    c = jax.lax.dot_general(a9t[...], b9[pl.ds(ni * _NT, _NT), :],
                            (((1,), (1,)), ((), ())),
                            preferred_element_type=jnp.float32)
    p48 = ni * _NT + jax.lax.broadcasted_iota(jnp.int32, (_MT, _NT), 1)
    bad = jnp.logical_or(p48 % _H >= _VALID, p48 >= _VALID * _H)
    c = jnp.where(bad, _NEG, c)

    for _ in range(_TOPK):
        m = jnp.max(c, axis=1, keepdims=True)
        sel = jnp.min(jnp.where(c >= m, p48, jnp.int32(2 ** 30)),
                      axis=1, keepdims=True)
        c = jnp.where(p48 == sel, _NEG, c)
        r0, r1, r2 = rv[:, 0:1], rv[:, 1:2], rv[:, 2:3]
        i0, i1, i2 = ri[:, 0:1], ri[:, 1:2], ri[:, 2:3]
        b0 = m > r0
        b1 = m > r1
        b2 = m > r2
        rv[:, 2:3] = jnp.where(b1, r1, jnp.where(b2, m, r2))
        ri[:, 2:3] = jnp.where(b1, i1, jnp.where(b2, sel, i2))
        rv[:, 1:2] = jnp.where(b0, r0, jnp.where(b1, m, r1))
        ri[:, 1:2] = jnp.where(b0, i0, jnp.where(b1, sel, i1))
        rv[:, 0:1] = jnp.where(b0, m, r0)
        ri[:, 0:1] = jnp.where(b0, sel, i0)

    @pl.when(ni == _NNI - 1)
    def _emit():
        val_ref[0] = rv[...] * nin_inv[...]
        idx_ref[0] = ri[...]


def _corr_topk(fi_dx, fr_dx):
    B = fi_dx.shape[0]
    grid = (B, _NMI, _NNI)
    return pl.pallas_call(
        _corr_topk_body,
        grid=grid,
        in_specs=[
            pl.BlockSpec((1, _PS, _PAD, _C), lambda b, mi, ni: (b, 0, 0, 0)),
            pl.BlockSpec((1, _PS, _PAD, _C), lambda b, mi, ni: (b, 0, 0, 0)),
        ],
        out_specs=[
            pl.BlockSpec((1, _MT, 8), lambda b, mi, ni: (b, mi, 0)),
            pl.BlockSpec((1, _MT, 8), lambda b, mi, ni: (b, mi, 0)),
        ],
        out_shape=[
            jax.ShapeDtypeStruct((B, _HW, 8), jnp.float32),
            jax.ShapeDtypeStruct((B, _HW, 8), jnp.int32),
        ],
        compiler_params=pltpu.CompilerParams(
            vmem_limit_bytes=100 * 1024 * 1024,
        ),
        scratch_shapes=[
            pltpu.VMEM((_HW, _K9), jnp.float32),
            pltpu.VMEM((_MT, _K9), jnp.float32),
            pltpu.VMEM((_PAD, 8), jnp.float32),
            pltpu.VMEM((_PAD, 8), jnp.float32),
            pltpu.VMEM((_PAD, 8), jnp.float32),
            pltpu.VMEM((_HW, 1), jnp.float32),
            pltpu.VMEM((_MT, 1), jnp.float32),
            pltpu.VMEM((_MT, 8), jnp.float32),
            pltpu.VMEM((_MT, 8), jnp.int32),
        ],
    )(fi_dx, fr_dx)


def _shift(x, i, j):
    if i == 0 and j == 0:
        return x
    nd = x.ndim
    pad = [(0, 0)] * nd
    pad[-3] = (i, 0)
    pad[-2] = (j, 0)
    h, w = x.shape[-3], x.shape[-2]
    return jnp.pad(x, pad)[..., :h, :w, :]


def _up2(x, axis_h, axis_w):
    return jnp.repeat(jnp.repeat(x, 2, axis=axis_h), 2, axis=axis_w)


def _vgg(img, p):
    mean = jnp.array([0.485, 0.456, 0.406], jnp.float32).reshape(1, 3, 1, 1)
    std = jnp.array([0.229, 0.224, 0.225], jnp.float32).reshape(1, 3, 1, 1)
    x = (img - mean) / std

    def conv(x, w, b):
        y = jax.lax.conv_general_dilated(x, w, (1, 1), 'SAME',
                                         dimension_numbers=('NCHW', 'OIHW', 'NCHW'))
        return y + b[None, :, None, None]

    def pool(x):
        return jax.lax.reduce_window(x, -jnp.inf, jax.lax.max,
                                     (1, 1, 2, 2), (1, 1, 2, 2), 'VALID')

    x = jax.nn.relu(conv(x, p['w11'], p['b11'])); r1 = x
    x = jax.nn.relu(conv(x, p['w12'], p['b12']))
    x = pool(x)
    x = jax.nn.relu(conv(x, p['w21'], p['b21'])); r2 = x
    x = jax.nn.relu(conv(x, p['w22'], p['b22']))
    x = pool(x)
    x = jax.nn.relu(conv(x, p['w31'], p['b31'])); r3 = x
    return r1, r2, r3


@jax.jit
def _run(dense_features1, dense_features2, img_ref_hr, vgg_params):
    B = dense_features1.shape[0]

    def prep(x):
        xt = jnp.transpose(x.reshape(B, _C, _HW), (0, 2, 1))
        xp = jnp.pad(xt, ((0, 0), (0, _PAD + 2 - _HW), (0, 0)))
        return jnp.stack([xp[:, dx:dx + _PAD, :] for dx in range(_PS)], axis=1)

    vals, idxs = _corr_topk(prep(dense_features1), prep(dense_features2))

    v = vals[:, :, :_TOPK].reshape(B, _H, _H, _TOPK)[:, :_VALID, :_VALID, :]
    v = jnp.transpose(v, (0, 3, 1, 2))
    ix = idxs[:, :, :_TOPK].reshape(B, _H, _H, _TOPK)[:, :_VALID, :_VALID, :]
    ix = jnp.transpose(ix, (0, 3, 1, 2))

    u = (ix % _H).astype(jnp.float32)
    vv = (ix // _H).astype(jnp.float32)
    gy = jnp.arange(_VALID, dtype=jnp.float32)[None, None, :, None]
    gx = jnp.arange(_VALID, dtype=jnp.float32)[None, None, None, :]
    flow = jnp.stack((u - gx, vv - gy), axis=-1)
    kf3 = jnp.pad(flow, ((0, 0), (0, 0), (0, 2), (0, 2), (0, 0)))
    ks3 = jnp.pad(v, ((0, 0), (0, 0), (1, 1), (1, 1)))

    ko3 = jnp.stack([_shift(kf3, i, j) for i in range(3) for j in range(3)], axis=2)
    kf2 = _up2(kf3, 2, 3) * 2.0
    ks2 = _up2(ks3, 2, 3)
    ko2 = jnp.stack([_shift(kf2, 2 * i, 2 * j) for i in range(3) for j in range(3)], axis=2)
    kf1 = _up2(_up2(kf3, 2, 3), 2, 3) * 4.0
    ks1 = _up2(_up2(ks3, 2, 3), 2, 3)
    ko1 = jnp.stack([_shift(kf1, 4 * i, 4 * j) for i in range(3) for j in range(3)], axis=2)

    r1, r2, r3 = _vgg(img_ref_hr, vgg_params)
    return (kf1, kf2, kf3, ko1, ko2, ko3, ks1, ks2, ks3, r1, r2, r3)


def kernel(dense_features1, dense_features2, img_ref_hr, vgg_params):
    return _run(dense_features1, dense_features2, img_ref_hr, vgg_params)


# speedup vs baseline: 8.5394x; 1.0014x over previous
import functools
import jax
import jax.numpy as jnp
from jax.experimental import pallas as pl
from jax.experimental.pallas import tpu as pltpu

_PS = 3
_TOPK = 3
_H = 48
_HW = _H * _H
_PAD = 2432
_C = 256
_K9 = _C * _PS * _PS
_MT = 128
_NT = 768
_NMI = _HW // _MT
_NNI = _HW // _NT
_VALID = _H - _PS + 1
_NEG = -1.0e30
_INIT = -3.0e38


def _corr_topk_body(fi_ref, fr_ref, val_ref, idx_ref,
                    b9, a9t, inv_i, inv_r, nin_inv, rv, ri):
    mi = pl.program_id(1)
    ni = pl.program_id(2)

    @pl.when(jnp.logical_and(mi == 0, ni == 0))
    def _batch_init():
        for dx in range(_PS):
            fi = fi_ref[0, dx]
            n = jnp.sqrt(jnp.sum(fi * fi, axis=1, keepdims=True))
            inv_i[:, dx:dx + 1] = jnp.maximum(n, 1e-12)
            fr = fr_ref[0, dx]
            m = jnp.sqrt(jnp.sum(fr * fr, axis=1, keepdims=True))
            inv_r[:, dx:dx + 1] = jnp.maximum(m, 1e-12)
        for dy in range(_PS):
            for dx in range(_PS):
                tap = dy * _PS + dx
                b9[:, pl.ds(tap * _C, _C)] = (
                    fr_ref[0, dx, pl.ds(dy * _H, _HW), :]
                    / inv_r[pl.ds(dy * _H, _HW), dx:dx + 1])
        bn = b9[...]
        rn = jnp.sqrt(jnp.sum(bn * bn, axis=1, keepdims=True))
        b9[...] = bn / (rn + 1e-5)

    @pl.when(ni == 0)
    def _mtile_init():
        m0 = pl.multiple_of(mi * _MT, _MT)
        for dy in range(_PS):
            for dx in range(_PS):
                tap = dy * _PS + dx
                a9t[:, pl.ds(tap * _C, _C)] = (
                    fi_ref[0, dx, pl.ds(m0 + dy * _H, _MT), :]
                    / inv_i[pl.ds(m0 + dy * _H, _MT), dx:dx + 1])
        a = a9t[...]
        nin_inv[...] = jnp.sqrt(jnp.sum(a * a, axis=1, keepdims=True)) + 1e-5
        rv[...] = jnp.full((_MT, 8), _INIT, jnp.float32)
        ri[...] = jnp.zeros((_MT, 8), jnp.int32)

    c = jax.lax.dot_general(a9t[...], b9[pl.ds(ni * _NT, _NT), :],
                            (((1,), (1,)), ((), ())),
                            preferred_element_type=jnp.float32)
    p48 = ni * _NT + jax.lax.broadcasted_iota(jnp.int32, (_MT, _NT), 1)
    bad = jnp.logical_or(p48 % _H >= _VALID, p48 >= _VALID * _H)
    c = jnp.where(bad, _NEG, c)

    for _ in range(_TOPK):
        m = jnp.max(c, axis=1, keepdims=True)
        sel = jnp.min(jnp.where(c >= m, p48, jnp.int32(2 ** 30)),
                      axis=1, keepdims=True)
        c = jnp.where(p48 == sel, _NEG, c)
        r0, r1, r2 = rv[:, 0:1], rv[:, 1:2], rv[:, 2:3]
        i0, i1, i2 = ri[:, 0:1], ri[:, 1:2], ri[:, 2:3]
        b0 = m > r0
        b1 = m > r1
        b2 = m > r2
        rv[:, 2:3] = jnp.where(b1, r1, jnp.where(b2, m, r2))
        ri[:, 2:3] = jnp.where(b1, i1, jnp.where(b2, sel, i2))
        rv[:, 1:2] = jnp.where(b0, r0, jnp.where(b1, m, r1))
        ri[:, 1:2] = jnp.where(b0, i0, jnp.where(b1, sel, i1))
        rv[:, 0:1] = jnp.where(b0, m, r0)
        ri[:, 0:1] = jnp.where(b0, sel, i0)

    @pl.when(ni == _NNI - 1)
    def _emit():
        val_ref[0] = rv[...] / nin_inv[...]
        idx_ref[0] = ri[...]


def _corr_topk(fi_dx, fr_dx):
    B = fi_dx.shape[0]
    grid = (B, _NMI, _NNI)
    return pl.pallas_call(
        _corr_topk_body,
        grid=grid,
        in_specs=[
            pl.BlockSpec((1, _PS, _PAD, _C), lambda b, mi, ni: (b, 0, 0, 0)),
            pl.BlockSpec((1, _PS, _PAD, _C), lambda b, mi, ni: (b, 0, 0, 0)),
        ],
        out_specs=[
            pl.BlockSpec((1, _MT, 8), lambda b, mi, ni: (b, mi, 0)),
            pl.BlockSpec((1, _MT, 8), lambda b, mi, ni: (b, mi, 0)),
        ],
        out_shape=[
            jax.ShapeDtypeStruct((B, _HW, 8), jnp.float32),
            jax.ShapeDtypeStruct((B, _HW, 8), jnp.int32),
        ],
        compiler_params=pltpu.CompilerParams(
            vmem_limit_bytes=100 * 1024 * 1024,
        ),
        scratch_shapes=[
            pltpu.VMEM((_HW, _K9), jnp.float32),
            pltpu.VMEM((_MT, _K9), jnp.float32),
            pltpu.VMEM((_PAD, 8), jnp.float32),
            pltpu.VMEM((_PAD, 8), jnp.float32),
            pltpu.VMEM((_MT, 1), jnp.float32),
            pltpu.VMEM((_MT, 8), jnp.float32),
            pltpu.VMEM((_MT, 8), jnp.int32),
        ],
    )(fi_dx, fr_dx)


def _shift(x, i, j):
    if i == 0 and j == 0:
        return x
    nd = x.ndim
    pad = [(0, 0)] * nd
    pad[-3] = (i, 0)
    pad[-2] = (j, 0)
    h, w = x.shape[-3], x.shape[-2]
    return jnp.pad(x, pad)[..., :h, :w, :]


def _up2(x, axis_h, axis_w):
    return jnp.repeat(jnp.repeat(x, 2, axis=axis_h), 2, axis=axis_w)


def _vgg(img, p):
    mean = jnp.array([0.485, 0.456, 0.406], jnp.float32).reshape(1, 3, 1, 1)
    std = jnp.array([0.229, 0.224, 0.225], jnp.float32).reshape(1, 3, 1, 1)
    x = (img - mean) / std

    def conv(x, w, b):
        y = jax.lax.conv_general_dilated(x, w, (1, 1), 'SAME',
                                         dimension_numbers=('NCHW', 'OIHW', 'NCHW'))
        return y + b[None, :, None, None]

    def pool(x):
        return jax.lax.reduce_window(x, -jnp.inf, jax.lax.max,
                                     (1, 1, 2, 2), (1, 1, 2, 2), 'VALID')

    x = jax.nn.relu(conv(x, p['w11'], p['b11'])); r1 = x
    x = jax.nn.relu(conv(x, p['w12'], p['b12']))
    x = pool(x)
    x = jax.nn.relu(conv(x, p['w21'], p['b21'])); r2 = x
    x = jax.nn.relu(conv(x, p['w22'], p['b22']))
    x = pool(x)
    x = jax.nn.relu(conv(x, p['w31'], p['b31'])); r3 = x
    return r1, r2, r3


@jax.jit
def _run(dense_features1, dense_features2, img_ref_hr, vgg_params):
    B = dense_features1.shape[0]

    def prep(x):
        xt = jnp.transpose(x.reshape(B, _C, _HW), (0, 2, 1))
        xp = jnp.pad(xt, ((0, 0), (0, _PAD + 2 - _HW), (0, 0)))
        return jnp.stack([xp[:, dx:dx + _PAD, :] for dx in range(_PS)], axis=1)

    vals, idxs = _corr_topk(prep(dense_features1), prep(dense_features2))

    v = vals[:, :, :_TOPK].reshape(B, _H, _H, _TOPK)[:, :_VALID, :_VALID, :]
    v = jnp.transpose(v, (0, 3, 1, 2))
    ix = idxs[:, :, :_TOPK].reshape(B, _H, _H, _TOPK)[:, :_VALID, :_VALID, :]
    ix = jnp.transpose(ix, (0, 3, 1, 2))

    u = (ix % _H).astype(jnp.float32)
    vv = (ix // _H).astype(jnp.float32)
    gy = jnp.arange(_VALID, dtype=jnp.float32)[None, None, :, None]
    gx = jnp.arange(_VALID, dtype=jnp.float32)[None, None, None, :]
    flow = jnp.stack((u - gx, vv - gy), axis=-1)
    kf3 = jnp.pad(flow, ((0, 0), (0, 0), (0, 2), (0, 2), (0, 0)))
    ks3 = jnp.pad(v, ((0, 0), (0, 0), (1, 1), (1, 1)))

    ko3 = jnp.stack([_shift(kf3, i, j) for i in range(3) for j in range(3)], axis=2)
    kf2 = _up2(kf3, 2, 3) * 2.0
    ks2 = _up2(ks3, 2, 3)
    ko2 = jnp.stack([_shift(kf2, 2 * i, 2 * j) for i in range(3) for j in range(3)], axis=2)
    kf1 = _up2(_up2(kf3, 2, 3), 2, 3) * 4.0
    ks1 = _up2(_up2(ks3, 2, 3), 2, 3)
    ko1 = jnp.stack([_shift(kf1, 4 * i, 4 * j) for i in range(3) for j in range(3)], axis=2)

    r1, r2, r3 = _vgg(img_ref_hr, vgg_params)
    return (kf1, kf2, kf3, ko1, ko2, ko3, ks1, ks2, ks3, r1, r2, r3)


def kernel(dense_features1, dense_features2, img_ref_hr, vgg_params):
    return _run(dense_features1, dense_features2, img_ref_hr, vgg_params)
